# Initial kernel scaffold; baseline (speedup 1.0000x reference)
import jax
import jax.numpy as jnp
from jax.experimental import pallas as pl


def kernel(x, pe):
    raise NotImplementedError("write your pallas kernel here")


# speedup vs baseline: 4.0116x; 4.0116x over previous
import functools

import jax
import jax.numpy as jnp
from jax import lax
from jax.experimental import pallas as pl
from jax.experimental.pallas import tpu as pltpu
from jax.experimental.pallas import tpu_sc as plsc

CH = 64
RPS = 128
NB = 4


def kernel(x, pe):
    b0, b1, _ = x.shape
    total = b0 * b1
    nchunks = total // RPS
    idx2d = x.astype(jnp.int32).reshape(nchunks, RPS)

    info = plsc.get_sparse_core_info()
    nw = info.num_cores * info.num_subcores
    cpw = nchunks // nw
    mesh = plsc.VectorSubcoreMesh(core_axis_name="c", subcore_axis_name="s")

    @functools.partial(
        pl.kernel,
        mesh=mesh,
        out_type=jax.ShapeDtypeStruct((total, CH), jnp.float32),
        compiler_params=pltpu.CompilerParams(use_tc_tiling_on_sc=False),
        scratch_types=[
            pltpu.VMEM((cpw, RPS), jnp.int32),
            pltpu.VMEM((NB, RPS, CH), jnp.float32),
            pltpu.SemaphoreType.DMA((NB,)),
            pltpu.SemaphoreType.DMA((NB,)),
        ],
    )
    def _lookup(idx_hbm, pe_hbm, out_hbm, idx_v, rows_v, gsem, ssem):
        wid = lax.axis_index("s") * info.num_cores + lax.axis_index("c")
        base = wid * cpw
        pltpu.sync_copy(idx_hbm.at[pl.ds(base, cpw)], idx_v)

        for b in range(NB):
            pltpu.async_copy(pe_hbm.at[idx_v.at[b]], rows_v.at[b], gsem.at[b])

        def body(o, _):
            for b in range(NB):
                g = o * NB + b
                pltpu.make_async_copy(
                    pe_hbm.at[idx_v.at[g]], rows_v.at[b], gsem.at[b]
                ).wait()
                dst = out_hbm.at[pl.ds((base + g) * RPS, RPS)]
                pltpu.async_copy(rows_v.at[b], dst, ssem.at[b])
                pltpu.make_async_copy(rows_v.at[b], dst, ssem.at[b]).wait()

                @pl.when(g + NB < cpw)
                def _():
                    pltpu.async_copy(
                        pe_hbm.at[idx_v.at[g + NB]], rows_v.at[b], gsem.at[b]
                    )

            return ()

        lax.fori_loop(0, cpw // NB, body, (), unroll=False)

    out = _lookup(idx2d, pe)
    return out.reshape(b0, b1, CH)


# speedup vs baseline: 4.9780x; 1.2409x over previous
import functools

import jax
import jax.numpy as jnp
from jax import lax
from jax.experimental import pallas as pl
from jax.experimental.pallas import tpu as pltpu
from jax.experimental.pallas import tpu_sc as plsc

CH = 64
RPS = 100
NB = 4


def kernel(x, pe):
    b0, b1, _ = x.shape
    halves = b1 // RPS
    nchunks = b0 * halves
    idx2d = x.astype(jnp.int32).reshape(nchunks, RPS)
    nrows = pe.shape[0]

    info = plsc.get_sparse_core_info()
    nw = info.num_cores * info.num_subcores
    cpw = nchunks // nw
    mesh = plsc.VectorSubcoreMesh(core_axis_name="c", subcore_axis_name="s")

    @functools.partial(
        pl.kernel,
        mesh=mesh,
        out_type=jax.ShapeDtypeStruct((b0, b1, CH), jnp.float32),
        compiler_params=pltpu.CompilerParams(use_tc_tiling_on_sc=False),
        scratch_types=[
            pltpu.VMEM((cpw, RPS), jnp.int32),
            pltpu.VMEM((NB, RPS, CH), jnp.float32),
            pltpu.VMEM_SHARED((nrows, CH), jnp.float32),
            pltpu.SemaphoreType.DMA((NB,)),
            pltpu.SemaphoreType.DMA((NB,)),
        ],
    )
    def _lookup(idx_hbm, pe_hbm, out_hbm, idx_v, rows_v, tbl_s, gsem, ssem):
        sid = lax.axis_index("s")
        wid = sid * info.num_cores + lax.axis_index("c")
        base = wid * cpw

        @pl.when(sid == 0)
        def _():
            pltpu.sync_copy(pe_hbm, tbl_s)

        pltpu.sync_copy(idx_hbm.at[pl.ds(base, cpw)], idx_v)
        plsc.subcore_barrier()

        def gather(g, b):
            return pltpu.make_async_copy(
                tbl_s.at[idx_v.at[g]], rows_v.at[b], gsem.at[b]
            )

        def scatter(g, b):
            c = base + g
            dst = out_hbm.at[c // halves, pl.ds((c % halves) * RPS, RPS)]
            return pltpu.make_async_copy(rows_v.at[b], dst, ssem.at[b])

        for b in range(NB):
            gather(b, b).start()

        def body(o, _):
            for b in range(NB):
                g = o * NB + b
                gather(g, b).wait()
                sc = scatter(g, b)
                sc.start()
                sc.wait()

                @pl.when(g + NB < cpw)
                def _():
                    gather(g + NB, b).start()

            return ()

        lax.fori_loop(0, cpw // NB, body, (), unroll=False)

    return _lookup(idx2d, pe)


# speedup vs baseline: 10.2315x; 2.0553x over previous
import functools

import jax
import jax.numpy as jnp
from jax import lax
from jax.experimental import pallas as pl
from jax.experimental.pallas import tpu as pltpu
from jax.experimental.pallas import tpu_sc as plsc

CH = 64
RPS = 100
NB = 4


def kernel(x, pe):
    b0, b1, _ = x.shape
    halves = b1 // RPS
    nchunks = b0 * halves
    idx2d = x.astype(jnp.int32).reshape(nchunks, RPS)
    nrows = pe.shape[0]

    info = plsc.get_sparse_core_info()
    nw = info.num_cores * info.num_subcores
    cpw = nchunks // nw
    mesh = plsc.VectorSubcoreMesh(core_axis_name="c", subcore_axis_name="s")

    @functools.partial(
        pl.kernel,
        mesh=mesh,
        out_type=jax.ShapeDtypeStruct((b0, b1, 2 * CH), jnp.float32),
        compiler_params=pltpu.CompilerParams(use_tc_tiling_on_sc=False),
        scratch_types=[
            pltpu.VMEM((cpw, RPS), jnp.int32),
            pltpu.VMEM((NB, RPS, CH), jnp.float32),
            pltpu.VMEM_SHARED((nrows, CH), jnp.float32),
            pltpu.SemaphoreType.DMA((NB,)),
            pltpu.SemaphoreType.DMA((NB,)),
        ],
    )
    def _lookup(idx_hbm, pe_hbm, out_hbm, idx_v, rows_v, tbl_s, gsem, ssem):
        sid = lax.axis_index("s")
        wid = sid * info.num_cores + lax.axis_index("c")
        base = wid * cpw

        @pl.when(sid == 0)
        def _():
            pltpu.sync_copy(pe_hbm, tbl_s)

        pltpu.sync_copy(idx_hbm.at[pl.ds(base, cpw)], idx_v)
        plsc.subcore_barrier()

        def gather(g, b):
            return pltpu.make_async_copy(
                tbl_s.at[idx_v.at[g]], rows_v.at[b], gsem.at[b]
            )

        def scatter(g, b):
            c = base + g
            dst = out_hbm.at[
                c // halves, pl.ds((c % halves) * RPS, RPS), pl.ds(0, CH)
            ]
            return pltpu.make_async_copy(rows_v.at[b], dst, ssem.at[b])

        for b in range(NB):
            gather(b, b).start()

        def body(o, _):
            for b in range(NB):
                g = o * NB + b
                gather(g, b).wait()
                sc = scatter(g, b)
                sc.start()
                sc.wait()

                @pl.when(g + NB < cpw)
                def _():
                    gather(g + NB, b).start()

            return ()

        lax.fori_loop(0, cpw // NB, body, (), unroll=False)

    return _lookup(idx2d, pe)[:, :, :CH]


# speedup vs baseline: 10.2386x; 1.0007x over previous
import functools

import jax
import jax.numpy as jnp
from jax import lax
from jax.experimental import pallas as pl
from jax.experimental.pallas import tpu as pltpu
from jax.experimental.pallas import tpu_sc as plsc

CH = 64
RPS = 100
NB = 8


def kernel(x, pe):
    b0, b1, _ = x.shape
    halves = b1 // RPS
    nchunks = b0 * halves
    idx2d = x.astype(jnp.int32).reshape(nchunks, RPS)
    nrows = pe.shape[0]

    info = plsc.get_sparse_core_info()
    nw = info.num_cores * info.num_subcores
    cpw = nchunks // nw
    mesh = plsc.VectorSubcoreMesh(core_axis_name="c", subcore_axis_name="s")

    @functools.partial(
        pl.kernel,
        mesh=mesh,
        out_type=jax.ShapeDtypeStruct((b0, b1, 2 * CH), jnp.float32),
        compiler_params=pltpu.CompilerParams(use_tc_tiling_on_sc=False),
        scratch_types=[
            pltpu.VMEM((cpw, RPS), jnp.int32),
            pltpu.VMEM((NB, RPS, CH), jnp.float32),
            pltpu.VMEM_SHARED((nrows, CH), jnp.float32),
            pltpu.SemaphoreType.DMA((NB,)),
            pltpu.SemaphoreType.DMA((NB,)),
        ],
    )
    def _lookup(idx_hbm, pe_hbm, out_hbm, idx_v, rows_v, tbl_s, gsem, ssem):
        sid = lax.axis_index("s")
        wid = sid * info.num_cores + lax.axis_index("c")
        base = wid * cpw

        @pl.when(sid == 0)
        def _():
            pltpu.sync_copy(pe_hbm, tbl_s)

        pltpu.sync_copy(idx_hbm.at[pl.ds(base, cpw)], idx_v)
        plsc.subcore_barrier()

        def gather(g, b):
            return pltpu.make_async_copy(
                tbl_s.at[idx_v.at[g]], rows_v.at[b], gsem.at[b]
            )

        def scatter(g, b):
            c = base + g
            dst = out_hbm.at[
                c // halves, pl.ds((c % halves) * RPS, RPS), pl.ds(0, CH)
            ]
            return pltpu.make_async_copy(rows_v.at[b], dst, ssem.at[b])

        for b in range(NB):
            gather(b, b).start()

        def body(o, _):
            for b in range(NB):
                g = o * NB + b
                gather(g, b).wait()
                sc = scatter(g, b)
                sc.start()
                sc.wait()

                @pl.when(g + NB < cpw)
                def _():
                    gather(g + NB, b).start()

            return ()

        lax.fori_loop(0, cpw // NB, body, (), unroll=False)

    return _lookup(idx2d, pe)[:, :, :CH]
